# baseline (speedup 1.0000x reference)
import functools

import jax
import jax.numpy as jnp
from jax import lax
from jax.experimental import pallas as pl
from jax.experimental.pallas import tpu as pltpu
from jax.experimental.pallas import tpu_sc as plsc

NUM_EMBEDDINGS = 1000000
EMB_D = 64
TSTEPS = 10
BATCH_B = 1024
SEQ_S = 20
NUM_WORKERS = 32
BBLK = 128
NBB = BATCH_B // BBLK
UNITS = SEQ_S * NBB
UNITS_PER_W = UNITS // NUM_WORKERS
LANES = 16
TPR = 8
WPT = EMB_D // 4
NROWS = NUM_EMBEDDINGS // TPR

_mesh = plsc.VectorSubcoreMesh(core_axis_name="c", subcore_axis_name="s")


@functools.partial(
    pl.kernel,
    mesh=_mesh,
    compiler_params=pltpu.CompilerParams(
        use_tc_tiling_on_sc=True, needs_layout_passes=False
    ),
    out_type=jax.ShapeDtypeStruct(
        (SEQ_S, TSTEPS, EMB_D // 8, NBB, 8, BBLK), jnp.float32
    ),
    scratch_types=[
        pltpu.VMEM((BBLK,), jnp.int32),
        pltpu.VMEM((BBLK,), jnp.int32),
        pltpu.VMEM((BBLK, TPR * WPT), jnp.int32),
        pltpu.VMEM((2, EMB_D // 8, 8, BBLK), jnp.float32),
        pltpu.SemaphoreType.DMA,
        pltpu.SemaphoreType.DMA,
    ],
)
def _spike_embed(
    ids_hbm, bits_hbm, out_hbm, idx_v, pr_v, rows_v, asm_v, sem_g, sem_o
):
    wid = lax.axis_index("s") * 2 + lax.axis_index("c")

    lane = lax.iota(jnp.int32, LANES)

    out_handles = []
    for u in range(UNITS_PER_W):
        buf = u % 2
        unit = wid * UNITS_PER_W + u
        s = unit // NBB
        b_blk = unit - s * NBB

        pltpu.sync_copy(ids_hbm.at[s, b_blk], idx_v)

        def shift(bc, carry):
            v = idx_v[pl.ds(bc * LANES, LANES)]
            pr_v[pl.ds(bc * LANES, LANES)] = lax.shift_right_logical(v, 3)
            return carry

        lax.fori_loop(0, BBLK // LANES, shift, 0)

        g = pltpu.make_async_copy(bits_hbm.at[pr_v], rows_v, sem_g)
        g.start()
        g.wait()

        if u >= 2:
            for h in out_handles[u - 2]:
                h.wait()

        def do_d(d, carry_d):
            db = d // 8
            di = d - db * 8
            w = d // 4
            sh = (d - w * 4) * 8
            wvec = jnp.broadcast_to(w, (LANES,))
            for bc in range(BBLK // LANES):
                idvec = idx_v[pl.ds(bc * LANES, LANES)]
                col = (idvec & 7) * WPT + wvec
                v = plsc.load_gather(rows_v, [bc * LANES + lane, col])
                bit = lax.shift_right_logical(v, sh) & 1
                asm_v[
                    buf, db, di, pl.ds(bc * LANES, LANES)
                ] = lax.convert_element_type(bit, jnp.float32)
            return carry_d

        lax.fori_loop(0, EMB_D, do_d, 0)

        hs = []
        for t in range(TSTEPS):
            c = pltpu.make_async_copy(
                asm_v.at[buf], out_hbm.at[s, t, :, b_blk], sem_o
            )
            c.start()
            hs.append(c)
        out_handles.append(hs)

    for hs in out_handles[-2:]:
        for h in hs:
            h.wait()


def kernel(token_ids, W, adaptive_threshold):
    ids = token_ids.astype(jnp.int32).T.reshape(SEQ_S, NBB, BBLK)
    spw = (
        W.reshape(NROWS, TPR, WPT, 4) > adaptive_threshold.astype(jnp.float32)
    ).astype(jnp.int32)
    bits = (
        spw[..., 0]
        | (spw[..., 1] << 8)
        | (spw[..., 2] << 16)
        | (spw[..., 3] << 24)
    ).reshape(NROWS, TPR * WPT)
    out6 = _spike_embed(ids, bits)
    return out6.transpose(3, 5, 0, 1, 2, 4).reshape(
        BATCH_B, SEQ_S, TSTEPS, EMB_D
    )


# speedup vs baseline: 8.3682x; 8.3682x over previous
import functools

import jax
import jax.numpy as jnp
from jax import lax
from jax.experimental import pallas as pl
from jax.experimental.pallas import tpu as pltpu
from jax.experimental.pallas import tpu_sc as plsc

NUM_EMBEDDINGS = 1000000
EMB_D = 64
TSTEPS = 10
BATCH_B = 1024
SEQ_S = 20
BBLK = 128
NBB = BATCH_B // BBLK
UNITS = SEQ_S * NBB
UNITS_PER_SUB = UNITS // 16
LANES = 16
DHALF = 32

_mesh = plsc.VectorSubcoreMesh(core_axis_name="c", subcore_axis_name="s")


@functools.partial(
    pl.kernel,
    mesh=_mesh,
    compiler_params=pltpu.CompilerParams(
        use_tc_tiling_on_sc=True, needs_layout_passes=False
    ),
    out_type=jax.ShapeDtypeStruct(
        (SEQ_S, TSTEPS, EMB_D // 8, NBB, 8, BBLK), jnp.float32
    ),
    scratch_types=[
        pltpu.VMEM((BBLK,), jnp.int32),
        pltpu.VMEM((BBLK,), jnp.int32),
        pltpu.VMEM((2, DHALF // 8, 8, BBLK), jnp.float32),
        pltpu.VMEM_SHARED((NUM_EMBEDDINGS,), jnp.int32),
        pltpu.SemaphoreType.DMA,
        pltpu.SemaphoreType.DMA,
    ],
)
def _spike_embed(
    ids_hbm, lo_hbm, hi_hbm, out_hbm, idx_v, wrd_v, asm_v, half_sp, sem_g, sem_o
):
    c = lax.axis_index("c")
    sub = lax.axis_index("s")

    @pl.when(sub == 0)
    def _load_half():
        @pl.when(c == 0)
        def _lo():
            pltpu.sync_copy(lo_hbm, half_sp)

        @pl.when(c == 1)
        def _hi():
            pltpu.sync_copy(hi_hbm, half_sp)

    plsc.subcore_barrier()

    lane = lax.iota(jnp.int32, LANES)

    out_handles = []
    for u in range(UNITS_PER_SUB):
        buf = u % 2
        unit = sub * UNITS_PER_SUB + u
        s = unit // NBB
        b_blk = unit - s * NBB

        pltpu.sync_copy(ids_hbm.at[s, b_blk], idx_v)
        g = pltpu.make_async_copy(half_sp.at[idx_v], wrd_v, sem_g)
        g.start()
        g.wait()

        if u >= 2:
            for h in out_handles[u - 2]:
                h.wait()

        def do_d(d, carry_d):
            db = d // 8
            di = d - db * 8
            for bc in range(BBLK // LANES):
                w = wrd_v[pl.ds(bc * LANES, LANES)]
                bit = lax.shift_right_logical(w, d) & 1
                asm_v[
                    buf, db, di, pl.ds(bc * LANES, LANES)
                ] = lax.convert_element_type(bit, jnp.float32)
            return carry_d

        lax.fori_loop(0, DHALF, do_d, 0)

        hs = []
        for t in range(TSTEPS):
            cp = pltpu.make_async_copy(
                asm_v.at[buf],
                out_hbm.at[s, t, pl.ds(c * (DHALF // 8), DHALF // 8), b_blk],
                sem_o,
            )
            cp.start()
            hs.append(cp)
        out_handles.append(hs)

    for hs in out_handles[-2:]:
        for h in hs:
            h.wait()


def kernel(token_ids, W, adaptive_threshold):
    ids = token_ids.astype(jnp.int32).T.reshape(SEQ_S, NBB, BBLK)
    thr = adaptive_threshold.astype(jnp.float32)
    wt = W.T
    halves = []
    for cix in range(2):
        acc = None
        for k in range(DHALF):
            b = (wt[cix * DHALF + k, :] > thr).astype(jnp.uint32) << k
            acc = b if acc is None else acc | b
        halves.append(acc.astype(jnp.int32))
    out6 = _spike_embed(ids, halves[0], halves[1])
    return out6.transpose(3, 5, 0, 1, 2, 4).reshape(
        BATCH_B, SEQ_S, TSTEPS, EMB_D
    )


# speedup vs baseline: 20.2987x; 2.4257x over previous
import functools

import jax
import jax.numpy as jnp
from jax import lax
from jax.experimental import pallas as pl
from jax.experimental.pallas import tpu as pltpu
from jax.experimental.pallas import tpu_sc as plsc

NUM_EMBEDDINGS = 1000000
EMB_D = 64
TSTEPS = 10
BATCH_B = 1024
SEQ_S = 20
NUM_WORKERS = 32
BBLK = 128
NBB = BATCH_B // BBLK
UNITS = SEQ_S * NBB
UNITS_PER_W = UNITS // NUM_WORKERS
LANES = 16

_mesh = plsc.VectorSubcoreMesh(core_axis_name="c", subcore_axis_name="s")


@functools.partial(
    pl.kernel,
    mesh=_mesh,
    compiler_params=pltpu.CompilerParams(
        use_tc_tiling_on_sc=True, needs_layout_passes=False
    ),
    out_type=jax.ShapeDtypeStruct(
        (SEQ_S, TSTEPS, EMB_D // 8, NBB, 8, BBLK), jnp.float32
    ),
    scratch_types=[
        pltpu.VMEM((BBLK,), jnp.int32),
        pltpu.VMEM((BBLK // 2, 8, EMB_D), jnp.float32),
        pltpu.VMEM((2, EMB_D // 8, 8, BBLK), jnp.float32),
        pltpu.VMEM((LANES,), jnp.float32),
        pltpu.SemaphoreType.DMA,
        pltpu.SemaphoreType.DMA,
    ],
)
def _spike_embed(
    ids_hbm, w_hbm, thr_hbm, out_hbm, idx_v, win_v, asm_v, thr_v, sem_g, sem_o
):
    wid = lax.axis_index("s") * 2 + lax.axis_index("c")

    pltpu.sync_copy(thr_hbm, thr_v)
    thr = thr_v[...]
    lane = lax.iota(jnp.int32, LANES)
    one = jnp.float32(1.0)
    zero = jnp.float32(0.0)

    out_handles = []
    for u in range(UNITS_PER_W):
        buf = u % 2
        unit = wid * UNITS_PER_W + u
        s = unit // NBB
        b_blk = unit - s * NBB

        pltpu.sync_copy(ids_hbm.at[s, b_blk], idx_v)

        if u >= 2:
            for h in out_handles[u - 2]:
                h.wait()

        for half in range(2):
            hb = half * (BBLK // 2)

            def fire(bc, carry):
                idvec = idx_v[pl.ds(hb + bc * LANES, LANES)]
                base8 = idvec & jnp.int32(~7)
                for l in range(LANES):
                    base = pl.multiple_of(base8[l], 8)
                    pltpu.make_async_copy(
                        w_hbm.at[pl.ds(base, 8), :],
                        win_v.at[bc * LANES + l],
                        sem_g,
                    ).start()
                return carry

            lax.fori_loop(0, BBLK // (2 * LANES), fire, 0)

            def drain(b, carry):
                pltpu.make_async_copy(
                    w_hbm.at[pl.ds(0, 8), :],
                    win_v.at[b],
                    sem_g,
                ).wait()
                return carry

            lax.fori_loop(0, BBLK // 2, drain, 0)

            def do_d(d, carry_d):
                db = d // 8
                di = d - db * 8
                dvec = jnp.broadcast_to(d, (LANES,))
                for bc in range(BBLK // (2 * LANES)):
                    idvec = idx_v[pl.ds(hb + bc * LANES, LANES)]
                    rowsel = idvec & 7
                    x = plsc.load_gather(
                        win_v, [bc * LANES + lane, rowsel, dvec]
                    )
                    asm_v[
                        buf, db, di, pl.ds(hb + bc * LANES, LANES)
                    ] = jnp.where(x > thr, one, zero)
                return carry_d

            lax.fori_loop(0, EMB_D, do_d, 0)

        hs = []
        for t in range(TSTEPS):
            c = pltpu.make_async_copy(
                asm_v.at[buf], out_hbm.at[s, t, :, b_blk], sem_o
            )
            c.start()
            hs.append(c)
        out_handles.append(hs)

    for hs in out_handles[-2:]:
        for h in hs:
            h.wait()


def kernel(token_ids, W, adaptive_threshold):
    ids = token_ids.astype(jnp.int32).T.reshape(SEQ_S, NBB, BBLK)
    thr16 = jnp.broadcast_to(adaptive_threshold.astype(jnp.float32), (LANES,))
    out6 = _spike_embed(ids, W, thr16)
    return out6.transpose(3, 5, 0, 1, 2, 4).reshape(
        BATCH_B, SEQ_S, TSTEPS, EMB_D
    )


# speedup vs baseline: 61.2623x; 3.0180x over previous
import functools

import jax
import jax.numpy as jnp
from jax import lax
from jax.experimental import pallas as pl
from jax.experimental.pallas import tpu as pltpu
from jax.experimental.pallas import tpu_sc as plsc

NUM_EMBEDDINGS = 1000000
EMB_D = 64
TSTEPS = 10
BATCH_B = 1024
SEQ_S = 20
BBLK = 128
NBB = BATCH_B // BBLK
UNITS = SEQ_S * NBB
UNITS_PER_SUB = UNITS // 16
LANES = 16
DHALF = 32

_mesh = plsc.VectorSubcoreMesh(core_axis_name="c", subcore_axis_name="s")


@functools.partial(
    pl.kernel,
    mesh=_mesh,
    compiler_params=pltpu.CompilerParams(
        use_tc_tiling_on_sc=True, needs_layout_passes=False
    ),
    out_type=jax.ShapeDtypeStruct(
        (SEQ_S, TSTEPS, EMB_D // 8, NBB, 8, BBLK), jnp.float32
    ),
    scratch_types=[
        pltpu.VMEM((BBLK,), jnp.int32),
        pltpu.VMEM((BBLK,), jnp.int32),
        pltpu.VMEM((2, DHALF // 8, 8, BBLK), jnp.float32),
        pltpu.VMEM_SHARED((NUM_EMBEDDINGS,), jnp.int32),
        pltpu.SemaphoreType.DMA,
        pltpu.SemaphoreType.DMA,
    ],
)
def _spike_embed(
    ids_hbm, lo_hbm, hi_hbm, out_hbm, idx_v, wrd_v, asm_v, half_sp, sem_g, sem_o
):
    c = lax.axis_index("c")
    sub = lax.axis_index("s")

    @pl.when(sub == 0)
    def _load_half():
        @pl.when(c == 0)
        def _lo():
            pltpu.sync_copy(lo_hbm, half_sp)

        @pl.when(c == 1)
        def _hi():
            pltpu.sync_copy(hi_hbm, half_sp)

    plsc.subcore_barrier()

    out_handles = []
    for u in range(UNITS_PER_SUB):
        buf = u % 2
        unit = sub * UNITS_PER_SUB + u
        s = unit // NBB
        b_blk = unit - s * NBB

        pltpu.sync_copy(ids_hbm.at[s, b_blk], idx_v)
        g = pltpu.make_async_copy(half_sp.at[idx_v], wrd_v, sem_g)
        g.start()
        g.wait()

        if u >= 2:
            for h in out_handles[u - 2]:
                h.wait()

        def do_d(d, carry_d):
            db = d // 8
            di = d - db * 8
            for bc in range(BBLK // LANES):
                w = wrd_v[pl.ds(bc * LANES, LANES)]
                bit = lax.shift_right_logical(w, d) & 1
                asm_v[
                    buf, db, di, pl.ds(bc * LANES, LANES)
                ] = lax.convert_element_type(bit, jnp.float32)
            return carry_d

        lax.fori_loop(0, DHALF, do_d, 0)

        hs = []
        for t in range(TSTEPS):
            cp = pltpu.make_async_copy(
                asm_v.at[buf],
                out_hbm.at[s, t, pl.ds(c * (DHALF // 8), DHALF // 8), b_blk],
                sem_o,
            )
            cp.start()
            hs.append(cp)
        out_handles.append(hs)

    for hs in out_handles[-2:]:
        for h in hs:
            h.wait()


def kernel(token_ids, W, adaptive_threshold):
    ids = token_ids.astype(jnp.int32).T.reshape(SEQ_S, NBB, BBLK)
    thr = adaptive_threshold.astype(jnp.float32)
    wv = W.T.reshape(2, DHALF, NUM_EMBEDDINGS)
    shifts = (jnp.uint32(1) << jnp.arange(DHALF, dtype=jnp.uint32))[:, None]
    lo = jnp.sum(
        (wv[0] > thr).astype(jnp.uint32) * shifts, axis=0, dtype=jnp.uint32
    ).astype(jnp.int32)
    hi = jnp.sum(
        (wv[1] > thr).astype(jnp.uint32) * shifts, axis=0, dtype=jnp.uint32
    ).astype(jnp.int32)
    out6 = _spike_embed(ids, lo, hi)
    return out6.transpose(3, 5, 0, 1, 2, 4).reshape(
        BATCH_B, SEQ_S, TSTEPS, EMB_D
    )
